# Initial kernel scaffold; baseline (speedup 1.0000x reference)
import jax
import jax.numpy as jnp
from jax.experimental import pallas as pl


def kernel(x, edge_index, edge_label_index, W1, b1, W2, b2):
    raise NotImplementedError("write your pallas kernel here")


# speedup vs baseline: 3.5561x; 3.5561x over previous
import functools

import jax
import jax.numpy as jnp
from jax import lax
from jax.experimental import pallas as pl
from jax.experimental.pallas import tpu as pltpu
from jax.experimental.pallas import tpu_sc as plsc

N_NODES = 10000
D_FEAT = 128
N_EDGES = 320000
N_LABEL = 100000

NC = 2
NS = 16
NW = NC * NS
CHUNK = 128
N_CHUNKS = N_EDGES // CHUNK
WBLK = 80
N_WBLK = N_NODES // WBLK

L_PAD = 102400
DEC_CHUNKS = L_PAD // CHUNK
DEC_PER_W = DEC_CHUNKS // NW

_mesh = plsc.VectorSubcoreMesh(core_axis_name="c", subcore_axis_name="s")

_SEG_SCRATCH = [
    pltpu.VMEM((CHUNK,), jnp.int32),
    pltpu.VMEM((CHUNK,), jnp.int32),
    pltpu.VMEM((CHUNK, D_FEAT), jnp.float32),
    pltpu.VMEM_SHARED((N_NODES, D_FEAT), jnp.float32),
    pltpu.SemaphoreType.DMA,
]


def _wb_loop(sid, src_ref, dst_ref):
    n_blk = N_WBLK // NS + jnp.where(sid < N_WBLK % NS, 1, 0)

    def wb_body(t, _):
        off = pl.multiple_of((sid + t * NS) * WBLK, 16)
        pltpu.sync_copy(src_ref.at[pl.ds(off, WBLK)], dst_ref.at[pl.ds(off, WBLK)])
        return 0

    lax.fori_loop(0, n_blk, wb_body, 0)


def _edge_loop(wid, fn):
    extra = N_CHUNKS % NW
    n_mine = N_CHUNKS // NW + jnp.where(wid < extra, 1, 0)

    def body(t, _):
        fn((wid + t * NW) * CHUNK)
        return 0

    lax.fori_loop(0, n_mine, body, 0)


def _agg_sum_body(x_hbm, src_hbm, dst_hbm, z2d_hbm, agg_out,
                  idx_s, idx_d, rows, acc, sem):
    cid = lax.axis_index("c")
    sid = lax.axis_index("s")
    wid = cid * NS + sid

    @pl.when(sid == 0)
    def _():
        pltpu.sync_copy(z2d_hbm, acc)

    plsc.subcore_barrier()

    def step(base):
        pltpu.sync_copy(src_hbm.at[pl.ds(base, CHUNK)], idx_s)
        pltpu.sync_copy(dst_hbm.at[pl.ds(base, CHUNK)], idx_d)
        pltpu.async_copy(x_hbm.at[idx_s], rows, sem).wait()
        pltpu.sync_copy(rows, acc.at[idx_d], add=True)

    _edge_loop(wid, step)
    plsc.subcore_barrier()
    _wb_loop(sid, acc, agg_out.at[cid])


_agg_sum = pl.kernel(
    _agg_sum_body,
    out_type=jax.ShapeDtypeStruct((NC, N_NODES, D_FEAT), jnp.float32),
    mesh=_mesh,
    scratch_types=_SEG_SCRATCH,
)


def _deg_sum_body(dst_hbm, z2d_hbm, ones_hbm, deg_out,
                  idx_d, ones_rows, acc, sem):
    cid = lax.axis_index("c")
    sid = lax.axis_index("s")
    wid = cid * NS + sid
    pltpu.sync_copy(ones_hbm, ones_rows)

    @pl.when(sid == 0)
    def _():
        pltpu.sync_copy(z2d_hbm, acc)

    plsc.subcore_barrier()

    def step(base):
        pltpu.sync_copy(dst_hbm.at[pl.ds(base, CHUNK)], idx_d)
        pltpu.sync_copy(ones_rows, acc.at[idx_d], add=True)

    _edge_loop(wid, step)
    plsc.subcore_barrier()
    _wb_loop(sid, acc, deg_out.at[cid])


_deg_sum = pl.kernel(
    _deg_sum_body,
    out_type=jax.ShapeDtypeStruct((NC, N_NODES, D_FEAT), jnp.float32),
    mesh=_mesh,
    scratch_types=[
        pltpu.VMEM((CHUNK,), jnp.int32),
        pltpu.VMEM((CHUNK, D_FEAT), jnp.float32),
        pltpu.VMEM_SHARED((N_NODES, D_FEAT), jnp.float32),
        pltpu.SemaphoreType.DMA,
    ],
)


def _decode_body(z_hbm, s_hbm, d_hbm, gs_out, gd_out,
                 idx_a, idx_b, rows_a, rows_b, sem):
    cid = lax.axis_index("c")
    sid = lax.axis_index("s")
    wid = cid * NS + sid

    def chunk_body(t, _):
        base = (wid * DEC_PER_W + t) * CHUNK
        pltpu.sync_copy(s_hbm.at[pl.ds(base, CHUNK)], idx_a)
        pltpu.sync_copy(d_hbm.at[pl.ds(base, CHUNK)], idx_b)
        pltpu.async_copy(z_hbm.at[idx_a], rows_a, sem).wait()
        pltpu.async_copy(z_hbm.at[idx_b], rows_b, sem).wait()
        pltpu.sync_copy(rows_a, gs_out.at[pl.ds(base, CHUNK)])
        pltpu.sync_copy(rows_b, gd_out.at[pl.ds(base, CHUNK)])
        return 0

    lax.fori_loop(0, DEC_PER_W, chunk_body, 0)


_decode = pl.kernel(
    _decode_body,
    out_type=(jax.ShapeDtypeStruct((L_PAD, D_FEAT), jnp.float32),
              jax.ShapeDtypeStruct((L_PAD, D_FEAT), jnp.float32)),
    mesh=_mesh,
    scratch_types=[
        pltpu.VMEM((CHUNK,), jnp.int32),
        pltpu.VMEM((CHUNK,), jnp.int32),
        pltpu.VMEM((CHUNK, D_FEAT), jnp.float32),
        pltpu.VMEM((CHUNK, D_FEAT), jnp.float32),
        pltpu.SemaphoreType.DMA,
    ],
)


def _rowsum_body(gs_ref, gd_ref, out_ref):
    out_ref[...] = jnp.sum(gs_ref[...] * gd_ref[...], axis=1)


def _rowsum_tc(gs, gd):
    rows_blk = 10240
    return pl.pallas_call(
        _rowsum_body,
        grid=(L_PAD // rows_blk,),
        in_specs=[pl.BlockSpec((rows_blk, D_FEAT), lambda i: (i, 0)),
                  pl.BlockSpec((rows_blk, D_FEAT), lambda i: (i, 0))],
        out_specs=pl.BlockSpec((rows_blk,), lambda i: (i,)),
        out_shape=jax.ShapeDtypeStruct((L_PAD,), jnp.float32),
    )(gs, gd)


def _layer_tc_body(relu, agg_ref, deg_ref, w_ref, b_ref, out_ref):
    part = agg_ref[0] + agg_ref[1]
    deg = jnp.sum(deg_ref[0] + deg_ref[1], axis=1, keepdims=True)
    deg = jnp.maximum(deg, 1.0)
    aggn = part / deg
    y = jnp.dot(aggn, w_ref[...], preferred_element_type=jnp.float32) + b_ref[...]
    if relu:
        y = jnp.maximum(y, 0.0)
    out_ref[...] = y


def _layer_tc(agg_part, deg_part, w, b, relu):
    rows_blk = 2000
    grid = (N_NODES // rows_blk,)
    return pl.pallas_call(
        functools.partial(_layer_tc_body, relu),
        grid=grid,
        in_specs=[
            pl.BlockSpec((NC, rows_blk, D_FEAT), lambda i: (0, i, 0)),
            pl.BlockSpec((NC, rows_blk, D_FEAT), lambda i: (0, i, 0)),
            pl.BlockSpec((D_FEAT, D_FEAT), lambda i: (0, 0)),
            pl.BlockSpec((1, D_FEAT), lambda i: (0, 0)),
        ],
        out_specs=pl.BlockSpec((rows_blk, D_FEAT), lambda i: (i, 0)),
        out_shape=jax.ShapeDtypeStruct((N_NODES, D_FEAT), jnp.float32),
    )(agg_part, deg_part, w, b)


def kernel(x, edge_index, edge_label_index, W1, b1, W2, b2):
    src = edge_index[0]
    dst = edge_index[1]
    s_pad = jnp.zeros((L_PAD,), jnp.int32).at[:N_LABEL].set(edge_label_index[0])
    d_pad = jnp.zeros((L_PAD,), jnp.int32).at[:N_LABEL].set(edge_label_index[1])
    z2d = jnp.zeros((N_NODES, D_FEAT), jnp.float32)
    ones2d = jnp.zeros((CHUNK, D_FEAT), jnp.float32).at[:, 0].set(1.0)

    degp = _deg_sum(dst, z2d, ones2d)
    agg1p = _agg_sum(x, src, dst, z2d)
    h = _layer_tc(agg1p, degp, W1, b1.reshape(1, D_FEAT), relu=True)
    agg2p = _agg_sum(h, src, dst, z2d)
    z = _layer_tc(agg2p, degp, W2, b2.reshape(1, D_FEAT), relu=False)
    gs, gd = _decode(z, s_pad, d_pad)
    scores_pad = _rowsum_tc(gs, gd)
    return scores_pad[:N_LABEL]
